# baseline (speedup 1.0000x reference)
import jax
import jax.numpy as jnp
from jax import lax
from jax.experimental import pallas as pl

_SEQ_LENS = (2048, 1024, 768, 512)
_HEADS = 16
_BLOCK_ROWS = {2048: 128, 1024: 256, 768: 256, 512: 512}

_F16_MAGIC = float(2.0 ** 112)
_F16_SUBNORM = float(2.0 ** -14)


def _decode_f16(h):
    t = lax.shift_left(h & 0x7FFF, 13)
    f = lax.bitcast_convert_type(t, jnp.float32) * _F16_MAGIC
    return jnp.where((h & 0x8000) != 0, -f, f)


def _encode_f16(p):
    b = lax.bitcast_convert_type(p, jnp.int32)
    rnd = (lax.shift_right_logical(b, 13) & 1) + 0xFFF
    hn = lax.shift_right_logical(b + rnd - 0x38000000, 13)
    hs = (p * 16777216.0 + 0.5).astype(jnp.int32)
    return jnp.where(p < _F16_SUBNORM, hs, hn)


def _softmax_block(x_ref, o_ref):
    b = x_ref[...]
    xlo = _decode_f16(b & 0xFFFF)
    xhi = _decode_f16(lax.shift_right_logical(b, 16))
    m = jnp.maximum(jnp.max(xlo, axis=-1, keepdims=True),
                    jnp.max(xhi, axis=-1, keepdims=True))
    elo = jnp.exp(xlo - m)
    ehi = jnp.exp(xhi - m)
    r = 1.0 / (jnp.sum(elo, axis=-1, keepdims=True) +
               jnp.sum(ehi, axis=-1, keepdims=True))
    hlo = _encode_f16(elo * r)
    hhi = _encode_f16(ehi * r)
    o_ref[...] = hlo | lax.shift_left(hhi, 16)


def kernel(x, seq_len, head_num):
    x32 = lax.bitcast_convert_type(x.reshape(-1, 2), jnp.int32)
    outs = []
    off = 0
    for s in _SEQ_LENS:
        n = _HEADS * s * s // 2
        rows = _HEADS * s
        c = s // 2
        seg = x32[off:off + n].reshape(rows, c)
        r = _BLOCK_ROWS[s]
        out = pl.pallas_call(
            _softmax_block,
            grid=(rows // r,),
            in_specs=[pl.BlockSpec((r, c), lambda i: (i, 0))],
            out_specs=pl.BlockSpec((r, c), lambda i: (i, 0)),
            out_shape=jax.ShapeDtypeStruct((rows, c), jnp.int32),
        )(seg)
        outs.append(out.reshape(-1))
        off += n
    flat = jnp.concatenate(outs)
    return lax.bitcast_convert_type(flat, jnp.float16).reshape(-1)


# speedup vs baseline: 34.1394x; 34.1394x over previous
import jax
import jax.numpy as jnp
from jax import lax
from jax.experimental import pallas as pl

_SEQ_LENS = (2048, 1024, 768, 512)
_HEADS = 16
_BLOCK_ROWS = {2048: 128, 1024: 256, 768: 256, 512: 512}

_F16_MAGIC = float(2.0 ** 112)
_F16_SUBNORM = float(2.0 ** -14)


def _decode_f16(h):
    t = lax.shift_left(h & 0x7FFF, 13)
    f = lax.bitcast_convert_type(t, jnp.float32) * _F16_MAGIC
    return jnp.where((h & 0x8000) != 0, -f, f)


def _encode_f16(p):
    b = lax.bitcast_convert_type(p, jnp.int32)
    rnd = (lax.shift_right_logical(b, 13) & 1) + 0xFFF
    hn = lax.shift_right_logical(b + rnd - 0x38000000, 13)
    hs = (p * 16777216.0 + 0.5).astype(jnp.int32)
    return jnp.where(p < _F16_SUBNORM, hs, hn)


def _softmax_halfrows(b):
    xlo = _decode_f16(b & 0xFFFF)
    xhi = _decode_f16(lax.shift_right_logical(b, 16))
    mlo = jnp.max(xlo, axis=-1, keepdims=True)
    mhi = jnp.max(xhi, axis=-1, keepdims=True)
    elo = jnp.exp(xlo - mlo)
    ehi = jnp.exp(xhi - mhi)
    rlo = 1.0 / jnp.sum(elo, axis=-1, keepdims=True)
    rhi = 1.0 / jnp.sum(ehi, axis=-1, keepdims=True)
    hlo = _encode_f16(elo * rlo)
    hhi = _encode_f16(ehi * rhi)
    return hlo | lax.shift_left(hhi, 16)


def _softmax_block(x_ref, o_ref):
    b = x_ref.bitcast(jnp.int32)[...]
    o_ref.bitcast(jnp.int32)[...] = _softmax_halfrows(b)


def kernel(x, seq_len, head_num):
    x16 = lax.bitcast_convert_type(x, jnp.bfloat16)
    outs = []
    off = 0
    for s in _SEQ_LENS:
        n = _HEADS * s * s
        rows = _HEADS * s
        seg = x16[off:off + n].reshape(rows, s)
        r = _BLOCK_ROWS[s]
        out = pl.pallas_call(
            _softmax_block,
            grid=(rows // r,),
            in_specs=[pl.BlockSpec((r, s), lambda i: (i, 0))],
            out_specs=pl.BlockSpec((r, s), lambda i: (i, 0)),
            out_shape=jax.ShapeDtypeStruct((rows, s), jnp.bfloat16),
        )(seg)
        outs.append(out.reshape(-1))
        off += n
    return lax.bitcast_convert_type(jnp.concatenate(outs), jnp.float16)
